# baseline (speedup 1.0000x reference)
import functools
import math

import jax
import jax.numpy as jnp
from jax import lax
from jax.experimental import pallas as pl
from jax.experimental.pallas import tpu as pltpu
from jax.experimental.pallas import tpu_sc as plsc

D_MODEL = 64
LANES = 16
NUM_CORES = 2
NUM_SUBCORES = 16
NUM_WORKERS = NUM_CORES * NUM_SUBCORES
SCALE = math.sqrt(D_MODEL)

B = 4096
L = 200
BT = B // 128
LT = L // 8
G = 2
N_ITEM = G * 128
JG = BT // G
N_ITEMS = L * JG
PER_W = N_ITEMS // NUM_WORKERS
OUT_LEN = B * L * D_MODEL


def _make_kernel():
    assert PER_W % 2 == 0
    mesh = plsc.VectorSubcoreMesh(core_axis_name="c", subcore_axis_name="s")

    scratch = (
        [pltpu.VMEM((N_ITEM,), jnp.int32) for _ in range(2)]
        + [pltpu.VMEM((N_ITEM, D_MODEL), jnp.float32) for _ in range(2)]
        + [pltpu.VMEM((N_ITEM * D_MODEL,), jnp.float32) for _ in range(2)]
        + [pltpu.SemaphoreType.DMA for _ in range(6)]
    )

    @functools.partial(
        pl.kernel,
        mesh=mesh,
        out_type=jax.ShapeDtypeStruct((OUT_LEN,), jnp.float32),
        scratch_types=scratch,
        compiler_params=pltpu.CompilerParams(
            use_tc_tiling_on_sc=False, needs_layout_passes=False),
    )
    def emb_kernel(x_hbm, table_hbm, out_hbm,
                   idx0, idx1, rows0, rows1, st0, st1,
                   isem0, isem1, gsem0, gsem1, osem0, osem1):
        idx = (idx0, idx1)
        rows = (rows0, rows1)
        stage = (st0, st1)
        isem = (isem0, isem1)
        gsem = (gsem0, gsem1)
        osem = (osem0, osem1)

        wid = lax.axis_index("s") * NUM_CORES + lax.axis_index("c")

        def item_coords(k):
            t = wid + k * NUM_WORKERS
            l = t // JG
            bt0 = (t % JG) * G
            return l, bt0

        def x_off(l, j):
            return ((l // 8) * BT + j) * (8 * 128) + (l % 8) * 128

        def idx_start(k, s):
            l, bt0 = item_coords(k)
            for g in range(G):
                pltpu.async_copy(
                    x_hbm.at[pl.ds(x_off(l, bt0 + g), 128)],
                    idx[s].at[pl.ds(g * 128, 128)], isem[s])

        def idx_wait(k, s):
            l, bt0 = item_coords(k)
            for g in range(G):
                pltpu.make_async_copy(
                    x_hbm.at[pl.ds(x_off(l, bt0 + g), 128)],
                    idx[s].at[pl.ds(g * 128, 128)], isem[s]).wait()

        def gather_start(s):
            pltpu.async_copy(table_hbm.at[idx[s]], rows[s], gsem[s])

        def gather_wait(s):
            pltpu.make_async_copy(
                table_hbm.at[idx[s]], rows[s], gsem[s]).wait()

        def out_off(l, bt0, dt):
            return ((l * 8 + dt) * BT + bt0) * (8 * 128)

        def store_start(k, s):
            l, bt0 = item_coords(k)
            for dt in range(8):
                pltpu.async_copy(
                    stage[s].at[pl.ds(dt * G * 1024, G * 1024)],
                    out_hbm.at[pl.ds(out_off(l, bt0, dt), G * 1024)],
                    osem[s])

        def store_wait(k, s):
            l, bt0 = item_coords(k)
            for dt in range(8):
                pltpu.make_async_copy(
                    stage[s].at[pl.ds(dt * G * 1024, G * 1024)],
                    out_hbm.at[pl.ds(out_off(l, bt0, dt), G * 1024)],
                    osem[s]).wait()

        def transpose_scale(s):
            r = rows[s]
            st = stage[s]

            def body(m, c):
                g = m // 8
                kk = m - g * 8
                rowv = lax.broadcasted_iota(jnp.int32, (LANES,), 0) \
                    + (g * 128 + kk * LANES)
                for dt in range(8):
                    for dr in range(8):
                        colv = jnp.full((LANES,), dt * 8 + dr, jnp.int32)
                        v = plsc.load_gather(r, [rowv, colv])
                        st[pl.ds(((dt * G + g) * 8 + dr) * 128 + kk * LANES,
                                 LANES)] = v * SCALE
                return c

            lax.fori_loop(0, G * 8, body, 0)

        idx_start(0, 0)
        idx_start(1, 1)
        idx_wait(0, 0)
        gather_start(0)

        def step(k, carry):
            idx_wait2 = idx_wait

            idx_wait2(2 * k + 1, 1)
            gather_start(1)

            gather_wait(0)

            @pl.when(k < PER_W // 2 - 1)
            def _():
                idx_start(2 * k + 2, 0)

            @pl.when(k > 0)
            def _():
                store_wait(2 * k - 2, 0)

            transpose_scale(0)
            store_start(2 * k, 0)

            gather_wait(1)

            @pl.when(k < PER_W // 2 - 1)
            def _():
                idx_start(2 * k + 3, 1)

            @pl.when(k > 0)
            def _():
                store_wait(2 * k - 1, 1)

            transpose_scale(1)
            store_start(2 * k + 1, 1)

            @pl.when(k < PER_W // 2 - 1)
            def _():
                idx_wait2(2 * k + 2, 0)
                gather_start(0)

            return carry

        lax.fori_loop(0, PER_W // 2, step, 0)

        store_wait(PER_W - 2, 0)
        store_wait(PER_W - 1, 1)

    return emb_kernel


@jax.jit
def kernel(x, table):
    x1d = (x.astype(jnp.int32).T
           .reshape(LT, 8, BT, 128).transpose(0, 2, 1, 3).reshape(-1))
    o1d = _make_kernel()(x1d, table)
    return (o1d.reshape(L, 8, BT, 8, 128)
            .transpose(2, 4, 0, 1, 3).reshape(B, L, D_MODEL))


# speedup vs baseline: 1.4222x; 1.4222x over previous
import functools
import math

import jax
import jax.numpy as jnp
from jax import lax
from jax.experimental import pallas as pl
from jax.experimental.pallas import tpu as pltpu
from jax.experimental.pallas import tpu_sc as plsc

D_MODEL = 64
LANES = 16
NUM_CORES = 2
NUM_SUBCORES = 16
NUM_WORKERS = NUM_CORES * NUM_SUBCORES
SCALE = math.sqrt(D_MODEL)

B = 4096
L = 200
BT = B // 128
LT = L // 8
G = 2
N_ITEM = G * 128
PER_W = L // 2
OUT_LEN = B * L * D_MODEL
IDX_ALL = LT * G * 8 * 128


def _make_kernel():
    mesh = plsc.VectorSubcoreMesh(core_axis_name="c", subcore_axis_name="s")

    scratch = (
        [pltpu.VMEM((IDX_ALL,), jnp.int32)]
        + [pltpu.VMEM((N_ITEM, D_MODEL), jnp.float32) for _ in range(2)]
        + [pltpu.VMEM((N_ITEM * D_MODEL,), jnp.float32) for _ in range(2)]
        + [pltpu.VMEM((4 * 16 * 17,), jnp.float32)]
        + [pltpu.SemaphoreType.DMA for _ in range(5)]
    )

    @functools.partial(
        pl.kernel,
        mesh=mesh,
        out_type=jax.ShapeDtypeStruct((OUT_LEN,), jnp.float32),
        scratch_types=scratch,
        compiler_params=pltpu.CompilerParams(
            use_tc_tiling_on_sc=False, needs_layout_passes=False),
    )
    def emb_kernel(x_hbm, table_hbm, out_hbm,
                   idx_all, rows0, rows1, st0, st1, sb,
                   isem, gsem0, gsem1, osem0, osem1):
        rows = (rows0, rows1)
        stage = (st0, st1)
        gsem = (gsem0, gsem1)
        osem = (osem0, osem1)

        wid = lax.axis_index("s") * NUM_CORES + lax.axis_index("c")
        base_l = wid // 16
        bt0 = (wid % 16) * G

        for lt in range(LT):
            for g in range(G):
                pltpu.async_copy(
                    x_hbm.at[pl.ds((lt * BT + bt0 + g) * 1024, 1024)],
                    idx_all.at[pl.ds((lt * G + g) * 1024, 1024)], isem)
        for _ in range(LT * G):
            pltpu.make_async_copy(
                x_hbm.at[pl.ds(0, 1024)], idx_all.at[pl.ds(0, 1024)],
                isem).wait()

        def item_l(k):
            return base_l + 2 * k

        def gather_start(k, s):
            l = item_l(k)
            lt = l // 8
            r = l - lt * 8
            for g in range(G):
                pltpu.async_copy(
                    table_hbm.at[idx_all.at[
                        pl.ds(((lt * G + g) * 8 + r) * 128, 128)]],
                    rows[s].at[pl.ds(g * 128, 128)], gsem[s])

        def gather_wait(k, s):
            l = item_l(k)
            lt = l // 8
            r = l - lt * 8
            for g in range(G):
                pltpu.make_async_copy(
                    table_hbm.at[idx_all.at[
                        pl.ds(((lt * G + g) * 8 + r) * 128, 128)]],
                    rows[s].at[pl.ds(g * 128, 128)], gsem[s]).wait()

        def out_off(l, dt):
            return ((l * 8 + dt) * BT + bt0) * 1024

        def store_start(k, s):
            l = item_l(k)
            for dt in range(8):
                pltpu.async_copy(
                    stage[s].at[pl.ds(dt * G * 1024, G * 1024)],
                    out_hbm.at[pl.ds(out_off(l, dt), G * 1024)],
                    osem[s])

        def store_wait(k, s):
            l = item_l(k)
            for dt in range(8):
                pltpu.make_async_copy(
                    stage[s].at[pl.ds(dt * G * 1024, G * 1024)],
                    out_hbm.at[pl.ds(out_off(l, dt), G * 1024)],
                    osem[s]).wait()

        def transpose_scale(s):
            r = rows[s]
            st = stage[s]
            iota17 = lax.broadcasted_iota(jnp.int32, (LANES,), 0) * 17

            def body(a, c):
                dyn = (a // 8) * 1024 + (a % 8) * 16
                row0 = a * 16
                for db in range(4):
                    for rr in range(16):
                        sb[pl.ds(db * 272 + rr * 17, 16)] = \
                            r[row0 + rr, pl.ds(db * 16, 16)]
                for db in range(4):
                    for cc in range(16):
                        d = db * 16 + cc
                        dt, dr = d // 8, d % 8
                        v = plsc.load_gather(
                            sb, [iota17 + (db * 272 + cc)])
                        st[pl.ds(dyn + dt * 2048 + dr * 128, 16)] = v * SCALE
                return c

            lax.fori_loop(0, 16, body, 0)

        gather_start(0, 0)

        def step(kk, carry):
            gather_start(2 * kk + 1, 1)
            gather_wait(2 * kk, 0)

            @pl.when(kk > 0)
            def _():
                store_wait(2 * kk - 2, 0)

            transpose_scale(0)
            store_start(2 * kk, 0)

            @pl.when(kk < PER_W // 2 - 1)
            def _():
                gather_start(2 * kk + 2, 0)

            gather_wait(2 * kk + 1, 1)

            @pl.when(kk > 0)
            def _():
                store_wait(2 * kk - 1, 1)

            transpose_scale(1)
            store_start(2 * kk + 1, 1)
            return carry

        lax.fori_loop(0, PER_W // 2, step, 0)

        store_wait(PER_W - 2, 0)
        store_wait(PER_W - 1, 1)

    return emb_kernel


@jax.jit
def kernel(x, table):
    x1d = (x.astype(jnp.int32).T
           .reshape(LT, 8, BT, 128).transpose(0, 2, 1, 3).reshape(-1))
    o1d = _make_kernel()(x1d, table)
    return (o1d.reshape(L, 8, BT, 8, 128)
            .transpose(2, 4, 0, 1, 3).reshape(B, L, D_MODEL))
